# Initial kernel scaffold; baseline (speedup 1.0000x reference)
import jax
import jax.numpy as jnp
from jax.experimental import pallas as pl


def kernel(agent_poses_his, agent_vels_his, pl_poses, x_pl, agent_classes, params):
    raise NotImplementedError("write your pallas kernel here")


# speedup vs baseline: 2.8603x; 2.8603x over previous
import math
import functools

import jax
import jax.numpy as jnp
from jax import lax
from jax.experimental import pallas as pl

H = 128
NH = 8
HD = 16
F = 64
NL = 2
KPL = 16
KAA = 16
B, N, T, M = 1, 32, 50, 64
TP = 56
BLK = 800
EPS = 1e-5
SCALE = 1.0 / math.sqrt(HD)

A2_KEYS = ['wq', 'bq', 'wk', 'bk', 'wv', 'bv', 'wrk', 'wrv', 'wo', 'bo',
           's1l', 'g1l', 'w1l', 'b1l', 'w2l', 'b2l', 's2l', 'g2l']
FO_KEYS = ['fr', 'w1c', 'w1s', 'w1l', 'b1', 's1', 'g1', 'w2', 'b2s',
           'so', 'go', 'wo', 'bo']


def _row(a):
    return a.reshape(1, -1)


def _k_ln(h, s_row, g_row):
    m = jnp.mean(h, axis=-1, keepdims=True)
    c = h - m
    v = jnp.mean(c * c, axis=-1, keepdims=True)
    return c * lax.rsqrt(v + EPS) * s_row + g_row


def _mm(a, b):
    return jnp.dot(a, b, preferred_element_type=jnp.float32)


def _heads_mat():
    j = lax.broadcasted_iota(jnp.int32, (H, NH), 0)
    hcol = lax.broadcasted_iota(jnp.int32, (H, NH), 1)
    return (j // HD == hcol).astype(jnp.float32)


def _fourier_body(*refs, d, has_cat):
    if has_cat:
        x_ref, cat_ref = refs[0], refs[1]
        w = refs[2:-1]
    else:
        x_ref = refs[0]
        cat_ref = None
        w = refs[1:-1]
    out_ref = refs[-1]
    p = dict(zip(FO_KEYS, w))
    acc = None
    for di in range(d):
        col = x_ref[:, di:di + 1]
        f = col * p['fr'][di:di + 1, :]
        h = (_mm(jnp.cos(f), p['w1c'][di]) + _mm(jnp.sin(f), p['w1s'][di])
             + col * p['w1l'][di:di + 1, :] + p['b1'][di:di + 1, :])
        h = jax.nn.relu(_k_ln(h, p['s1'][di:di + 1, :], p['g1'][di:di + 1, :]))
        t = _mm(h, p['w2'][di])
        acc = t if acc is None else acc + t
    acc = acc + p['b2s'][...]
    if cat_ref is not None:
        acc = acc + cat_ref[...]
    acc = jax.nn.relu(_k_ln(acc, p['so'][...], p['go'][...]))
    out_ref[...] = _mm(acc, p['wo'][...]) + p['bo'][...]


def _fourier_params(p, d):
    w1 = p['w1']
    return {
        'fr': p['freqs'] * (2.0 * jnp.pi),
        'w1c': w1[:, :F, :],
        'w1s': w1[:, F:2 * F, :],
        'w1l': w1[:, 2 * F, :],
        'b1': p['b1'], 's1': p['s1'], 'g1': p['g1'],
        'w2': p['w2'],
        'b2s': _row(jnp.sum(p['b2'], axis=0)),
        'so': _row(p['so']), 'go': _row(p['go']),
        'wo': p['wo'], 'bo': _row(p['bo']),
    }


def _fourier(p, x_rows, cat_rows=None):
    P, d = x_rows.shape
    assert P % BLK == 0, P
    wp = _fourier_params(p, d)
    ws = [wp[k] for k in FO_KEYS]
    full = lambda a: pl.BlockSpec(a.shape, lambda i: (0,) * a.ndim)
    in_specs = [pl.BlockSpec((BLK, d), lambda i: (i, 0))]
    args = [x_rows]
    if cat_rows is not None:
        in_specs.append(pl.BlockSpec((BLK, H), lambda i: (i, 0)))
        args.append(cat_rows)
    in_specs += [full(a) for a in ws]
    args += ws
    return pl.pallas_call(
        functools.partial(_fourier_body, d=d, has_cat=cat_rows is not None),
        grid=(P // BLK,),
        in_specs=in_specs,
        out_specs=pl.BlockSpec((BLK, H), lambda i: (i, 0)),
        out_shape=jax.ShapeDtypeStruct((P, H), jnp.float32),
    )(*args)


def _a2t_body(xt_ref, rt_ref, *w_refs):
    out_ref = w_refs[-1]
    p = dict(zip(A2_KEYS, w_refs[:-1]))
    S = _heads_mat()
    xt = xt_ref[0]
    q = _mm(xt, p['wq'][...]) + p['bq'][...]
    k0 = _mm(xt, p['wk'][...]) + p['bk'][...]
    v0 = _mm(xt, p['wv'][...]) + p['bv'][...]
    rt2 = rt_ref[0]
    rk = _mm(rt2, p['wrk'][...]).reshape(TP, TP, H)
    rv = _mm(rt2, p['wrv'][...]).reshape(TP, TP, H)
    kf = rk + k0[None, :, :]
    vf = rv + v0[None, :, :]
    prod = q[:, None, :] * kf
    sc = _mm(prod.reshape(TP * TP, H), S).reshape(TP, TP, NH) * SCALE
    tk = lax.broadcasted_iota(jnp.int32, (TP, TP, NH), 1)
    sc = jnp.where(tk < T, sc, -1e30)
    mx = jnp.max(sc, axis=1, keepdims=True)
    e = jnp.exp(sc - mx)
    a = e / jnp.sum(e, axis=1, keepdims=True)
    aexp = _mm(a.reshape(TP * TP, NH), S.T).reshape(TP, TP, H)
    o = jnp.sum(aexp * vf, axis=1)
    x1 = _k_ln(xt + _mm(o, p['wo'][...]) + p['bo'][...], p['s1l'][...], p['g1l'][...])
    hdn = _mm(jax.nn.relu(_mm(x1, p['w1l'][...]) + p['b1l'][...]), p['w2l'][...]) + p['b2l'][...]
    out_ref[0] = _k_ln(x1 + hdn, p['s2l'][...], p['g2l'][...])


def _a2t(p, xt_pad, rt2):
    ws = [p[k] if p[k].ndim == 2 else _row(p[k]) for k in A2_KEYS]
    full = lambda a: pl.BlockSpec(a.shape, lambda i: (0,) * a.ndim)
    return pl.pallas_call(
        _a2t_body,
        grid=(B * N,),
        in_specs=[pl.BlockSpec((1, TP, H), lambda i: (i, 0, 0)),
                  pl.BlockSpec((1, TP * TP, H), lambda i: (i, 0, 0))]
                 + [full(a) for a in ws],
        out_specs=pl.BlockSpec((1, TP, H), lambda i: (i, 0, 0)),
        out_shape=jax.ShapeDtypeStruct((B * N, TP, H), jnp.float32),
    )(xt_pad, rt2, *ws)


def _attend(xs, table, idx, rp, p, S, tbl_sz, K):
    oh = (idx[:, :, None] == lax.broadcasted_iota(jnp.int32, (N, K, tbl_sz), 2))
    kvg = _mm(oh.astype(jnp.float32).reshape(N * K, tbl_sz), table)
    kf = _mm(kvg, p['wk'][...]) + _mm(rp, p['wrk'][...]) + p['bk'][...]
    vf = (_mm(kvg, p['wv'][...]) + _mm(rp, p['wrv'][...]) + p['bv'][...]).reshape(N, K, H)
    q = _mm(xs, p['wq'][...]) + p['bq'][...]
    prod = q[:, None, :] * kf.reshape(N, K, H)
    sc = _mm(prod.reshape(N * K, H), S).reshape(N, K, NH) * SCALE
    mx = jnp.max(sc, axis=1, keepdims=True)
    e = jnp.exp(sc - mx)
    a = e / jnp.sum(e, axis=1, keepdims=True)
    aexp = _mm(a.reshape(N * K, NH), S.T).reshape(N, K, H)
    o = jnp.sum(aexp * vf, axis=1)
    x1 = _k_ln(xs + _mm(o, p['wo'][...]) + p['bo'][...], p['s1l'][...], p['g1l'][...])
    hdn = _mm(jax.nn.relu(_mm(x1, p['w1l'][...]) + p['b1l'][...]), p['w2l'][...]) + p['b2l'][...]
    return _k_ln(x1 + hdn, p['s2l'][...], p['g2l'][...])


def _a2s_body(xs_ref, xpl_ref, knnpl_ref, rpl_ref, knnaa_ref, raa_ref, *w_refs):
    out_ref = w_refs[-1]
    ppl = dict(zip(A2_KEYS, w_refs[:len(A2_KEYS)]))
    paa = dict(zip(A2_KEYS, w_refs[len(A2_KEYS):2 * len(A2_KEYS)]))
    S = _heads_mat()
    xs = xs_ref[0]
    xs1 = _attend(xs, xpl_ref[...], knnpl_ref[0], rpl_ref[0], ppl, S, M, KPL)
    xs2 = _attend(xs1, xs1, knnaa_ref[0], raa_ref[0], paa, S, N, KAA)
    out_ref[0] = xs2


def _a2s(ppl, paa, xs, xpl, knn_pl, rpl_rows, knn_aa, raa_rows):
    ws = ([ppl[k] if ppl[k].ndim == 2 else _row(ppl[k]) for k in A2_KEYS]
          + [paa[k] if paa[k].ndim == 2 else _row(paa[k]) for k in A2_KEYS])
    full = lambda a: pl.BlockSpec(a.shape, lambda i: (0,) * a.ndim)
    return pl.pallas_call(
        _a2s_body,
        grid=(B * T,),
        in_specs=[pl.BlockSpec((1, N, H), lambda i: (i, 0, 0)),
                  full(xpl),
                  pl.BlockSpec((1, N, KPL), lambda i: (i, 0, 0)),
                  pl.BlockSpec((1, N * KPL, H), lambda i: (i, 0, 0)),
                  pl.BlockSpec((1, N, KAA), lambda i: (i, 0, 0)),
                  pl.BlockSpec((1, N * KAA, H), lambda i: (i, 0, 0))]
                 + [full(a) for a in ws],
        out_specs=pl.BlockSpec((1, N, H), lambda i: (i, 0, 0)),
        out_shape=jax.ShapeDtypeStruct((B * T, N, H), jnp.float32),
    )(xs, xpl, knn_pl, rpl_rows, knn_aa, raa_rows, *ws)


def _ang(ctr, nbr):
    return jnp.arctan2(ctr[..., 0] * nbr[..., 1] - ctr[..., 1] * nbr[..., 0],
                       jnp.sum(ctr * nbr, axis=-1))


def _wrap(a):
    return (a + jnp.pi) % (2.0 * jnp.pi) - jnp.pi


def kernel(agent_poses_his, agent_vels_his, pl_poses, x_pl, agent_classes, params):
    prm = params
    poses = agent_poses_his
    pos = poses[..., :2]
    hd = poses[..., 2]
    motion = jnp.concatenate(
        [jnp.zeros((B, N, 1, 2), poses.dtype), pos[:, :, 1:] - pos[:, :, :-1]], axis=2)
    hvec = jnp.stack([jnp.cos(hd), jnp.sin(hd)], axis=-1)
    cont = jnp.stack([
        jnp.linalg.norm(motion, axis=-1),
        _ang(hvec, motion),
        jnp.linalg.norm(agent_vels_his, axis=-1),
        _ang(hvec, agent_vels_his),
    ], axis=-1)
    cat = prm['type_emb'][agent_classes][:, :, None, :]
    cat_rows = jnp.broadcast_to(cat, (B, N, T, H)).reshape(B * N * T, H)
    x = _fourier(prm['x_a'], cont.reshape(B * N * T, 4), cat_rows).reshape(B, N, T, H)

    plp = pl_poses[..., :2]
    plh = pl_poses[..., 2]
    rel_all = pos[:, :, :, None, :] - plp[:, None, None, :, :]
    d2_pl = jnp.sum(rel_all * rel_all, axis=-1)
    _, idx = lax.top_k(-d2_pl, KPL)
    knn_pl = jnp.transpose(idx, (0, 2, 1, 3)).reshape(B * T, N, KPL)
    knn_tn = knn_pl.reshape(B, T, N, KPL)
    pos_t = jnp.transpose(pos, (0, 2, 1, 3))
    hd_t = jnp.transpose(hd, (0, 2, 1))
    hvec_t = jnp.transpose(hvec, (0, 2, 1, 3))
    plp_g = plp[0][knn_tn]
    plh_g = plh[0][knn_tn]
    rel_g = pos_t[:, :, :, None, :] - plp_g
    d_g = jnp.linalg.norm(rel_g, axis=-1)
    r_pl_raw = jnp.stack([
        d_g,
        _ang(hvec_t[:, :, :, None, :], rel_g),
        _wrap(plh_g - hd_t[..., None]),
    ], axis=-1)
    rpl_rows = _fourier(prm['r_a2pl'], r_pl_raw.reshape(-1, 3)).reshape(B * T, N * KPL, H)

    rel_aa_all = pos_t[:, :, :, None, :] - pos_t[:, :, None, :, :]
    d2_aa = jnp.sum(rel_aa_all * rel_aa_all, axis=-1)
    _, idx2 = lax.top_k(-d2_aa, KAA)
    knn_aa = idx2.reshape(B * T, N, KAA)
    pos_g = jnp.take_along_axis(
        jnp.broadcast_to(pos_t[:, :, None, :, :], (B, T, N, N, 2)), idx2[..., None], axis=3)
    hd_g = jnp.take_along_axis(
        jnp.broadcast_to(hd_t[:, :, None, :], (B, T, N, N)), idx2, axis=3)
    rel_g2 = pos_t[:, :, :, None, :] - pos_g
    d_g2 = jnp.linalg.norm(rel_g2, axis=-1)
    r_aa_raw = jnp.stack([
        d_g2,
        _ang(hvec_t[:, :, :, None, :], rel_g2),
        _wrap(hd_t[..., None] - hd_g),
    ], axis=-1)
    raa_rows = _fourier(prm['r_a2a'], r_aa_raw.reshape(-1, 3)).reshape(B * T, N * KAA, H)

    rel_t = pos[:, :, :, None, :] - pos[:, :, None, :, :]
    rel_hv = hvec[:, :, :, None, :] - hvec[:, :, None, :, :]
    ti = jnp.arange(-T + 1, 1)
    rel_ti = jnp.broadcast_to((ti[:, None] - ti[None, :]).astype(poses.dtype), (B, N, T, T))
    r_t_raw = jnp.stack([
        jnp.linalg.norm(rel_t, axis=-1),
        _ang(rel_hv, rel_t),
        _wrap(hd[:, :, :, None] - hd[:, :, None, :]),
        rel_ti,
    ], axis=-1)
    r_t = _fourier(prm['r_t'], r_t_raw.reshape(-1, 4)).reshape(B * N, T, T, H)
    rt2 = jnp.pad(r_t, ((0, 0), (0, TP - T), (0, TP - T), (0, 0))).reshape(B * N, TP * TP, H)

    xpl_t = x_pl[0]
    for i in range(NL):
        xt = jnp.pad(x.reshape(B * N, T, H), ((0, 0), (0, TP - T), (0, 0)))
        xt = _a2t(prm['a2t'][i], xt, rt2)[:, :T]
        xs = jnp.transpose(xt.reshape(B, N, T, H), (0, 2, 1, 3)).reshape(B * T, N, H)
        xs = _a2s(prm['a2pl'][i], prm['a2a'][i], xs, xpl_t,
                  knn_pl, rpl_rows, knn_aa, raa_rows)
        x = jnp.transpose(xs.reshape(B, T, N, H), (0, 2, 1, 3))
    return x


# speedup vs baseline: 3.9674x; 1.3871x over previous
import math
import functools

import jax
import jax.numpy as jnp
from jax import lax
from jax.experimental import pallas as pl

H = 128
NH = 8
HD = 16
F = 64
NL = 2
KPL = 16
KAA = 16
B, N, T, M = 1, 32, 50, 64
TP = 56
BLK = 800
EPS = 1e-5
SCALE = 1.0 / math.sqrt(HD)

A2_KEYS = ['wq', 'bq', 'wk', 'bk', 'wv', 'bv', 'wrk', 'wrv', 'wo', 'bo',
           's1l', 'g1l', 'w1l', 'b1l', 'w2l', 'b2l', 's2l', 'g2l']
FO_KEYS = ['fr', 'w1c', 'w1s', 'w1l', 'b1', 's1', 'g1', 'w2', 'b2s',
           'so', 'go', 'wo', 'bo']


def _row(a):
    return a.reshape(1, -1)


def _k_ln(h, s_row, g_row):
    m = jnp.mean(h, axis=-1, keepdims=True)
    c = h - m
    v = jnp.mean(c * c, axis=-1, keepdims=True)
    return c * lax.rsqrt(v + EPS) * s_row + g_row


def _mm(a, b):
    return jnp.dot(a, b, preferred_element_type=jnp.float32)


_COS_C = (0.9999999922855516, -19.739205552336067, 64.93917213578796,
          -85.45116383102751, 60.176212682457745, -26.000455681229646,
          6.575502264034935)
_SIN_C = (6.28318530388885, -41.34170085507124, 81.60515474468119,
          -76.70345298880159, 42.02959370037914, -14.913885622758668,
          3.25815356852333)


def _sincos2pi(y):
    r = y - jnp.floor(y + 0.5)
    u = r * r
    c = jnp.float32(_COS_C[6])
    s = jnp.float32(_SIN_C[6])
    for k in (5, 4, 3, 2, 1, 0):
        c = c * u + jnp.float32(_COS_C[k])
        s = s * u + jnp.float32(_SIN_C[k])
    return c, s * r


def _heads_mat():
    j = lax.broadcasted_iota(jnp.int32, (H, NH), 0)
    hcol = lax.broadcasted_iota(jnp.int32, (H, NH), 1)
    return (j // HD == hcol).astype(jnp.float32)


def _fourier_body(*refs, d, has_cat):
    if has_cat:
        x_ref, cat_ref = refs[0], refs[1]
        w = refs[2:-1]
    else:
        x_ref = refs[0]
        cat_ref = None
        w = refs[1:-1]
    out_ref = refs[-1]
    p = dict(zip(FO_KEYS, w))
    acc = None
    for di in range(d):
        col = x_ref[:, di:di + 1]
        cf, sf = _sincos2pi(col * p['fr'][di:di + 1, :])
        h = (_mm(cf, p['w1c'][di]) + _mm(sf, p['w1s'][di])
             + col * p['w1l'][di:di + 1, :] + p['b1'][di:di + 1, :])
        h = jax.nn.relu(_k_ln(h, p['s1'][di:di + 1, :], p['g1'][di:di + 1, :]))
        t = _mm(h, p['w2'][di])
        acc = t if acc is None else acc + t
    acc = acc + p['b2s'][...]
    if cat_ref is not None:
        acc = acc + cat_ref[...]
    acc = jax.nn.relu(_k_ln(acc, p['so'][...], p['go'][...]))
    out_ref[...] = _mm(acc, p['wo'][...]) + p['bo'][...]


def _fourier_params(p, d):
    w1 = p['w1']
    return {
        'fr': p['freqs'],
        'w1c': w1[:, :F, :],
        'w1s': w1[:, F:2 * F, :],
        'w1l': w1[:, 2 * F, :],
        'b1': p['b1'], 's1': p['s1'], 'g1': p['g1'],
        'w2': p['w2'],
        'b2s': _row(jnp.sum(p['b2'], axis=0)),
        'so': _row(p['so']), 'go': _row(p['go']),
        'wo': p['wo'], 'bo': _row(p['bo']),
    }


def _fourier(p, x_rows, cat_rows=None):
    P, d = x_rows.shape
    assert P % BLK == 0, P
    wp = _fourier_params(p, d)
    ws = [wp[k] for k in FO_KEYS]
    full = lambda a: pl.BlockSpec(a.shape, lambda i: (0,) * a.ndim)
    in_specs = [pl.BlockSpec((BLK, d), lambda i: (i, 0))]
    args = [x_rows]
    if cat_rows is not None:
        in_specs.append(pl.BlockSpec((BLK, H), lambda i: (i, 0)))
        args.append(cat_rows)
    in_specs += [full(a) for a in ws]
    args += ws
    return pl.pallas_call(
        functools.partial(_fourier_body, d=d, has_cat=cat_rows is not None),
        grid=(P // BLK,),
        in_specs=in_specs,
        out_specs=pl.BlockSpec((BLK, H), lambda i: (i, 0)),
        out_shape=jax.ShapeDtypeStruct((P, H), jnp.float32),
    )(*args)


def _a2t_body(xt_ref, rt_ref, *w_refs):
    out_ref = w_refs[-1]
    p = dict(zip(A2_KEYS, w_refs[:-1]))
    S = _heads_mat()
    xt = xt_ref[0]
    q = _mm(xt, p['wq'][...]) + p['bq'][...]
    k0 = _mm(xt, p['wk'][...]) + p['bk'][...]
    v0 = _mm(xt, p['wv'][...]) + p['bv'][...]
    rt2 = rt_ref[0]
    rk = _mm(rt2, p['wrk'][...]).reshape(TP, TP, H)
    rv = _mm(rt2, p['wrv'][...]).reshape(TP, TP, H)
    kf = rk + k0[None, :, :]
    vf = rv + v0[None, :, :]
    prod = q[:, None, :] * kf
    sc = _mm(prod.reshape(TP * TP, H), S).reshape(TP, TP, NH) * SCALE
    tk = lax.broadcasted_iota(jnp.int32, (TP, TP, NH), 1)
    sc = jnp.where(tk < T, sc, -1e30)
    mx = jnp.max(sc, axis=1, keepdims=True)
    e = jnp.exp(sc - mx)
    a = e / jnp.sum(e, axis=1, keepdims=True)
    aexp = _mm(a.reshape(TP * TP, NH), S.T).reshape(TP, TP, H)
    o = jnp.sum(aexp * vf, axis=1)
    x1 = _k_ln(xt + _mm(o, p['wo'][...]) + p['bo'][...], p['s1l'][...], p['g1l'][...])
    hdn = _mm(jax.nn.relu(_mm(x1, p['w1l'][...]) + p['b1l'][...]), p['w2l'][...]) + p['b2l'][...]
    out_ref[0] = _k_ln(x1 + hdn, p['s2l'][...], p['g2l'][...])


def _a2t(p, xt_pad, rt2):
    ws = [p[k] if p[k].ndim == 2 else _row(p[k]) for k in A2_KEYS]
    full = lambda a: pl.BlockSpec(a.shape, lambda i: (0,) * a.ndim)
    return pl.pallas_call(
        _a2t_body,
        grid=(B * N,),
        in_specs=[pl.BlockSpec((1, TP, H), lambda i: (i, 0, 0)),
                  pl.BlockSpec((1, TP * TP, H), lambda i: (i, 0, 0))]
                 + [full(a) for a in ws],
        out_specs=pl.BlockSpec((1, TP, H), lambda i: (i, 0, 0)),
        out_shape=jax.ShapeDtypeStruct((B * N, TP, H), jnp.float32),
    )(xt_pad, rt2, *ws)


def _attend(xs, table, idx, rp, p, S, tbl_sz, K):
    oh = (idx[:, :, None] == lax.broadcasted_iota(jnp.int32, (N, K, tbl_sz), 2))
    kvg = _mm(oh.astype(jnp.float32).reshape(N * K, tbl_sz), table)
    kf = _mm(kvg, p['wk'][...]) + _mm(rp, p['wrk'][...]) + p['bk'][...]
    vf = (_mm(kvg, p['wv'][...]) + _mm(rp, p['wrv'][...]) + p['bv'][...]).reshape(N, K, H)
    q = _mm(xs, p['wq'][...]) + p['bq'][...]
    prod = q[:, None, :] * kf.reshape(N, K, H)
    sc = _mm(prod.reshape(N * K, H), S).reshape(N, K, NH) * SCALE
    mx = jnp.max(sc, axis=1, keepdims=True)
    e = jnp.exp(sc - mx)
    a = e / jnp.sum(e, axis=1, keepdims=True)
    aexp = _mm(a.reshape(N * K, NH), S.T).reshape(N, K, H)
    o = jnp.sum(aexp * vf, axis=1)
    x1 = _k_ln(xs + _mm(o, p['wo'][...]) + p['bo'][...], p['s1l'][...], p['g1l'][...])
    hdn = _mm(jax.nn.relu(_mm(x1, p['w1l'][...]) + p['b1l'][...]), p['w2l'][...]) + p['b2l'][...]
    return _k_ln(x1 + hdn, p['s2l'][...], p['g2l'][...])


def _a2s_body(xs_ref, xpl_ref, knnpl_ref, rpl_ref, knnaa_ref, raa_ref, *w_refs):
    out_ref = w_refs[-1]
    ppl = dict(zip(A2_KEYS, w_refs[:len(A2_KEYS)]))
    paa = dict(zip(A2_KEYS, w_refs[len(A2_KEYS):2 * len(A2_KEYS)]))
    S = _heads_mat()
    xs = xs_ref[0]
    xs1 = _attend(xs, xpl_ref[...], knnpl_ref[0], rpl_ref[0], ppl, S, M, KPL)
    xs2 = _attend(xs1, xs1, knnaa_ref[0], raa_ref[0], paa, S, N, KAA)
    out_ref[0] = xs2


def _a2s(ppl, paa, xs, xpl, knn_pl, rpl_rows, knn_aa, raa_rows):
    ws = ([ppl[k] if ppl[k].ndim == 2 else _row(ppl[k]) for k in A2_KEYS]
          + [paa[k] if paa[k].ndim == 2 else _row(paa[k]) for k in A2_KEYS])
    full = lambda a: pl.BlockSpec(a.shape, lambda i: (0,) * a.ndim)
    return pl.pallas_call(
        _a2s_body,
        grid=(B * T,),
        in_specs=[pl.BlockSpec((1, N, H), lambda i: (i, 0, 0)),
                  full(xpl),
                  pl.BlockSpec((1, N, KPL), lambda i: (i, 0, 0)),
                  pl.BlockSpec((1, N * KPL, H), lambda i: (i, 0, 0)),
                  pl.BlockSpec((1, N, KAA), lambda i: (i, 0, 0)),
                  pl.BlockSpec((1, N * KAA, H), lambda i: (i, 0, 0))]
                 + [full(a) for a in ws],
        out_specs=pl.BlockSpec((1, N, H), lambda i: (i, 0, 0)),
        out_shape=jax.ShapeDtypeStruct((B * T, N, H), jnp.float32),
    )(xs, xpl, knn_pl, rpl_rows, knn_aa, raa_rows, *ws)


def _ang(ctr, nbr):
    return jnp.arctan2(ctr[..., 0] * nbr[..., 1] - ctr[..., 1] * nbr[..., 0],
                       jnp.sum(ctr * nbr, axis=-1))


def _wrap(a):
    return (a + jnp.pi) % (2.0 * jnp.pi) - jnp.pi


def kernel(agent_poses_his, agent_vels_his, pl_poses, x_pl, agent_classes, params):
    prm = params
    poses = agent_poses_his
    pos = poses[..., :2]
    hd = poses[..., 2]
    motion = jnp.concatenate(
        [jnp.zeros((B, N, 1, 2), poses.dtype), pos[:, :, 1:] - pos[:, :, :-1]], axis=2)
    hvec = jnp.stack([jnp.cos(hd), jnp.sin(hd)], axis=-1)
    cont = jnp.stack([
        jnp.linalg.norm(motion, axis=-1),
        _ang(hvec, motion),
        jnp.linalg.norm(agent_vels_his, axis=-1),
        _ang(hvec, agent_vels_his),
    ], axis=-1)
    cat = prm['type_emb'][agent_classes][:, :, None, :]
    cat_rows = jnp.broadcast_to(cat, (B, N, T, H)).reshape(B * N * T, H)
    x = _fourier(prm['x_a'], cont.reshape(B * N * T, 4), cat_rows).reshape(B, N, T, H)

    plp = pl_poses[..., :2]
    plh = pl_poses[..., 2]
    rel_all = pos[:, :, :, None, :] - plp[:, None, None, :, :]
    d2_pl = jnp.sum(rel_all * rel_all, axis=-1)
    _, idx = lax.top_k(-jnp.sqrt(d2_pl), KPL)
    knn_pl = jnp.transpose(idx, (0, 2, 1, 3)).reshape(B * T, N, KPL)
    knn_tn = knn_pl.reshape(B, T, N, KPL)
    pos_t = jnp.transpose(pos, (0, 2, 1, 3))
    hd_t = jnp.transpose(hd, (0, 2, 1))
    hvec_t = jnp.transpose(hvec, (0, 2, 1, 3))
    plp_g = plp[0][knn_tn]
    plh_g = plh[0][knn_tn]
    rel_g = pos_t[:, :, :, None, :] - plp_g
    d_g = jnp.linalg.norm(rel_g, axis=-1)
    r_pl_raw = jnp.stack([
        d_g,
        _ang(hvec_t[:, :, :, None, :], rel_g),
        _wrap(plh_g - hd_t[..., None]),
    ], axis=-1)
    rpl_rows = _fourier(prm['r_a2pl'], r_pl_raw.reshape(-1, 3)).reshape(B * T, N * KPL, H)

    rel_aa_all = pos_t[:, :, :, None, :] - pos_t[:, :, None, :, :]
    d2_aa = jnp.sum(rel_aa_all * rel_aa_all, axis=-1)
    _, idx2 = lax.top_k(-jnp.sqrt(d2_aa), KAA)
    knn_aa = idx2.reshape(B * T, N, KAA)
    pos_g = jnp.take_along_axis(
        jnp.broadcast_to(pos_t[:, :, None, :, :], (B, T, N, N, 2)), idx2[..., None], axis=3)
    hd_g = jnp.take_along_axis(
        jnp.broadcast_to(hd_t[:, :, None, :], (B, T, N, N)), idx2, axis=3)
    rel_g2 = pos_t[:, :, :, None, :] - pos_g
    d_g2 = jnp.linalg.norm(rel_g2, axis=-1)
    r_aa_raw = jnp.stack([
        d_g2,
        _ang(hvec_t[:, :, :, None, :], rel_g2),
        _wrap(hd_t[..., None] - hd_g),
    ], axis=-1)
    raa_rows = _fourier(prm['r_a2a'], r_aa_raw.reshape(-1, 3)).reshape(B * T, N * KAA, H)

    rel_t = pos[:, :, :, None, :] - pos[:, :, None, :, :]
    rel_hv = hvec[:, :, :, None, :] - hvec[:, :, None, :, :]
    ti = jnp.arange(-T + 1, 1)
    rel_ti = jnp.broadcast_to((ti[:, None] - ti[None, :]).astype(poses.dtype), (B, N, T, T))
    r_t_raw = jnp.stack([
        jnp.linalg.norm(rel_t, axis=-1),
        _ang(rel_hv, rel_t),
        _wrap(hd[:, :, :, None] - hd[:, :, None, :]),
        rel_ti,
    ], axis=-1)
    r_t = _fourier(prm['r_t'], r_t_raw.reshape(-1, 4)).reshape(B * N, T, T, H)
    rt2 = jnp.pad(r_t, ((0, 0), (0, TP - T), (0, TP - T), (0, 0))).reshape(B * N, TP * TP, H)

    xpl_t = x_pl[0]
    for i in range(NL):
        xt = jnp.pad(x.reshape(B * N, T, H), ((0, 0), (0, TP - T), (0, 0)))
        xt = _a2t(prm['a2t'][i], xt, rt2)[:, :T]
        xs = jnp.transpose(xt.reshape(B, N, T, H), (0, 2, 1, 3)).reshape(B * T, N, H)
        xs = _a2s(prm['a2pl'][i], prm['a2a'][i], xs, xpl_t,
                  knn_pl, rpl_rows, knn_aa, raa_rows)
        x = jnp.transpose(xs.reshape(B, T, N, H), (0, 2, 1, 3))
    return x


# speedup vs baseline: 4.4325x; 1.1172x over previous
import math
import functools

import jax
import jax.numpy as jnp
from jax import lax
from jax.experimental import pallas as pl
from jax.experimental.pallas import tpu as pltpu
from jax.experimental.pallas import tpu_sc as plsc

H = 128
NH = 8
HD = 16
F = 64
NL = 2
KPL = 16
KAA = 16
B, N, T, M = 1, 32, 50, 64
TP = 56
BLK = 800
EPS = 1e-5
SCALE = 1.0 / math.sqrt(HD)

A2_KEYS = ['wq', 'bq', 'wk', 'bk', 'wv', 'bv', 'wrk', 'wrv', 'wo', 'bo',
           's1l', 'g1l', 'w1l', 'b1l', 'w2l', 'b2l', 's2l', 'g2l']
FO_KEYS = ['fr', 'w1c', 'w1s', 'w1l', 'b1', 's1', 'g1', 'w2', 'b2s',
           'so', 'go', 'wo', 'bo']


def _row(a):
    return a.reshape(1, -1)


def _k_ln(h, s_row, g_row):
    m = jnp.mean(h, axis=-1, keepdims=True)
    c = h - m
    v = jnp.mean(c * c, axis=-1, keepdims=True)
    return c * lax.rsqrt(v + EPS) * s_row + g_row


def _mm(a, b):
    return jnp.dot(a, b, preferred_element_type=jnp.float32)


_COS_C = (0.9999999922855516, -19.739205552336067, 64.93917213578796,
          -85.45116383102751, 60.176212682457745, -26.000455681229646,
          6.575502264034935)
_SIN_C = (6.28318530388885, -41.34170085507124, 81.60515474468119,
          -76.70345298880159, 42.02959370037914, -14.913885622758668,
          3.25815356852333)


def _sincos2pi(y):
    r = y - jnp.floor(y + 0.5)
    u = r * r
    c = jnp.float32(_COS_C[6])
    s = jnp.float32(_SIN_C[6])
    for k in (5, 4, 3, 2, 1, 0):
        c = c * u + jnp.float32(_COS_C[k])
        s = s * u + jnp.float32(_SIN_C[k])
    return c, s * r


def _heads_mat():
    j = lax.broadcasted_iota(jnp.int32, (H, NH), 0)
    hcol = lax.broadcasted_iota(jnp.int32, (H, NH), 1)
    return (j // HD == hcol).astype(jnp.float32)


def _branch(x, p, di):
    cf, sf = _sincos2pi(x * p['fr'][di:di + 1, :])
    h = (_mm(cf, p['w1c'][di]) + _mm(sf, p['w1s'][di])
         + x * p['w1l'][di:di + 1, :] + p['b1'][di:di + 1, :])
    h = jax.nn.relu(_k_ln(h, p['s1'][di:di + 1, :], p['g1'][di:di + 1, :]))
    return _mm(h, p['w2'][di])


def _fourier_core(x, p, d, add):
    acc = None
    for di in range(d):
        t = _branch(x[:, di:di + 1], p, di)
        acc = t if acc is None else acc + t
    acc = acc + p['b2s'][...]
    if add is not None:
        acc = acc + add
    acc = jax.nn.relu(_k_ln(acc, p['so'][...], p['go'][...]))
    return _mm(acc, p['wo'][...]) + p['bo'][...]


def _fourier_body(*refs, d, has_cat):
    if has_cat:
        x_ref, cat_ref = refs[0], refs[1]
        w = refs[2:-1]
    else:
        x_ref = refs[0]
        cat_ref = None
        w = refs[1:-1]
    out_ref = refs[-1]
    p = dict(zip(FO_KEYS, w))
    add = cat_ref[...] if cat_ref is not None else None
    out_ref[...] = _fourier_core(x_ref[...], p, d, add)


def _fourier_params(p, d):
    w1 = p['w1']
    return {
        'fr': p['freqs'],
        'w1c': w1[:, :F, :],
        'w1s': w1[:, F:2 * F, :],
        'w1l': w1[:, 2 * F, :],
        'b1': p['b1'], 's1': p['s1'], 'g1': p['g1'],
        'w2': p['w2'],
        'b2s': _row(jnp.sum(p['b2'], axis=0)),
        'so': _row(p['so']), 'go': _row(p['go']),
        'wo': p['wo'], 'bo': _row(p['bo']),
    }


def _fourier(p, x_rows, cat_rows=None):
    P, d = x_rows.shape
    assert P % BLK == 0, P
    wp = _fourier_params(p, d)
    ws = [wp[k] for k in FO_KEYS]
    full = lambda a: pl.BlockSpec(a.shape, lambda i: (0,) * a.ndim)
    in_specs = [pl.BlockSpec((BLK, d), lambda i: (i, 0))]
    args = [x_rows]
    if cat_rows is not None:
        in_specs.append(pl.BlockSpec((BLK, H), lambda i: (i, 0)))
        args.append(cat_rows)
    in_specs += [full(a) for a in ws]
    args += ws
    return pl.pallas_call(
        functools.partial(_fourier_body, d=d, has_cat=cat_rows is not None),
        grid=(P // BLK,),
        in_specs=in_specs,
        out_specs=pl.BlockSpec((BLK, H), lambda i: (i, 0)),
        out_shape=jax.ShapeDtypeStruct((P, H), jnp.float32),
    )(*args)


def _fourier_g_body(x_ref, add_ref, *refs, d):
    out_ref = refs[-1]
    p = dict(zip(FO_KEYS, refs[:-1]))
    out_ref[0] = _fourier_core(x_ref[0], p, d, add_ref[...])


def _fourier_grouped(p, x_g, add_rows, d_used):
    G, R, d = x_g.shape
    wp = _fourier_params(p, None)
    ws = [wp[k] for k in FO_KEYS]
    full = lambda a: pl.BlockSpec(a.shape, lambda i: (0,) * a.ndim)
    return pl.pallas_call(
        functools.partial(_fourier_g_body, d=d_used),
        grid=(G,),
        in_specs=[pl.BlockSpec((1, R, d), lambda i: (i, 0, 0)),
                  pl.BlockSpec((R, H), lambda i: (0, 0))]
                 + [full(a) for a in ws],
        out_specs=pl.BlockSpec((1, R, H), lambda i: (i, 0, 0)),
        out_shape=jax.ShapeDtypeStruct((G, R, H), jnp.float32),
    )(x_g, add_rows, *ws)


def _branch_body(x_ref, *refs, di):
    out_ref = refs[-1]
    p = dict(zip(FO_KEYS, refs[:-1]))
    out_ref[...] = _branch(x_ref[...], p, di)


def _fourier_one_branch(p, x_rows, di):
    R = x_rows.shape[0]
    wp = _fourier_params(p, None)
    ws = [wp[k] for k in FO_KEYS]
    full = lambda a: pl.BlockSpec(a.shape, lambda i: (0,) * a.ndim)
    return pl.pallas_call(
        functools.partial(_branch_body, di=di),
        grid=(1,),
        in_specs=[pl.BlockSpec((R, 1), lambda i: (0, 0))]
                 + [full(a) for a in ws],
        out_specs=pl.BlockSpec((R, H), lambda i: (0, 0)),
        out_shape=jax.ShapeDtypeStruct((R, H), jnp.float32),
    )(x_rows, *ws)


def _a2t_body(xt_ref, rt_ref, *w_refs):
    out_ref = w_refs[-1]
    p = dict(zip(A2_KEYS, w_refs[:-1]))
    S = _heads_mat()
    xt = xt_ref[0]
    q = _mm(xt, p['wq'][...]) + p['bq'][...]
    k0 = _mm(xt, p['wk'][...]) + p['bk'][...]
    v0 = _mm(xt, p['wv'][...]) + p['bv'][...]
    rt2 = rt_ref[0]
    rk = _mm(rt2, p['wrk'][...]).reshape(T, TP, H)
    rv = _mm(rt2, p['wrv'][...]).reshape(T, TP, H)
    kf = rk + k0[None, :, :]
    vf = rv + v0[None, :, :]
    prod = q[:T][:, None, :] * kf
    sc = _mm(prod.reshape(T * TP, H), S).reshape(T, TP, NH) * SCALE
    tk = lax.broadcasted_iota(jnp.int32, (T, TP, NH), 1)
    sc = jnp.where(tk < T, sc, -1e30)
    mx = jnp.max(sc, axis=1, keepdims=True)
    e = jnp.exp(sc - mx)
    a = e / jnp.sum(e, axis=1, keepdims=True)
    aexp = _mm(a.reshape(T * TP, NH), S.T).reshape(T, TP, H)
    o = jnp.sum(aexp * vf, axis=1)
    x1 = _k_ln(xt[:T] + _mm(o, p['wo'][...]) + p['bo'][...], p['s1l'][...], p['g1l'][...])
    hdn = _mm(jax.nn.relu(_mm(x1, p['w1l'][...]) + p['b1l'][...]), p['w2l'][...]) + p['b2l'][...]
    out_ref[0, :T, :] = _k_ln(x1 + hdn, p['s2l'][...], p['g2l'][...])


def _a2t(p, xt_pad, rt2):
    ws = [p[k] if p[k].ndim == 2 else _row(p[k]) for k in A2_KEYS]
    full = lambda a: pl.BlockSpec(a.shape, lambda i: (0,) * a.ndim)
    return pl.pallas_call(
        _a2t_body,
        grid=(B * N,),
        in_specs=[pl.BlockSpec((1, TP, H), lambda i: (i, 0, 0)),
                  pl.BlockSpec((1, T * TP, H), lambda i: (i, 0, 0))]
                 + [full(a) for a in ws],
        out_specs=pl.BlockSpec((1, TP, H), lambda i: (i, 0, 0)),
        out_shape=jax.ShapeDtypeStruct((B * N, TP, H), jnp.float32),
    )(xt_pad, rt2, *ws)


def _attend(xs, table, idx, rp, p, S, tbl_sz, K):
    oh = (idx[:, :, None] == lax.broadcasted_iota(jnp.int32, (N, K, tbl_sz), 2))
    kvg = _mm(oh.astype(jnp.float32).reshape(N * K, tbl_sz), table)
    kf = _mm(kvg, p['wk'][...]) + _mm(rp, p['wrk'][...]) + p['bk'][...]
    vf = (_mm(kvg, p['wv'][...]) + _mm(rp, p['wrv'][...]) + p['bv'][...]).reshape(N, K, H)
    q = _mm(xs, p['wq'][...]) + p['bq'][...]
    prod = q[:, None, :] * kf.reshape(N, K, H)
    sc = _mm(prod.reshape(N * K, H), S).reshape(N, K, NH) * SCALE
    mx = jnp.max(sc, axis=1, keepdims=True)
    e = jnp.exp(sc - mx)
    a = e / jnp.sum(e, axis=1, keepdims=True)
    aexp = _mm(a.reshape(N * K, NH), S.T).reshape(N, K, H)
    o = jnp.sum(aexp * vf, axis=1)
    x1 = _k_ln(xs + _mm(o, p['wo'][...]) + p['bo'][...], p['s1l'][...], p['g1l'][...])
    hdn = _mm(jax.nn.relu(_mm(x1, p['w1l'][...]) + p['b1l'][...]), p['w2l'][...]) + p['b2l'][...]
    return _k_ln(x1 + hdn, p['s2l'][...], p['g2l'][...])


def _a2s_body(xs_ref, xpl_ref, knnpl_ref, rpl_ref, knnaa_ref, raa_ref, *w_refs):
    out_ref = w_refs[-1]
    ppl = dict(zip(A2_KEYS, w_refs[:len(A2_KEYS)]))
    paa = dict(zip(A2_KEYS, w_refs[len(A2_KEYS):2 * len(A2_KEYS)]))
    S = _heads_mat()
    xs = xs_ref[0]
    xs1 = _attend(xs, xpl_ref[...], knnpl_ref[0], rpl_ref[0], ppl, S, M, KPL)
    xs2 = _attend(xs1, xs1, knnaa_ref[0], raa_ref[0], paa, S, N, KAA)
    out_ref[0] = xs2


def _a2s(ppl, paa, xs, xpl, knn_pl, rpl_rows, knn_aa, raa_rows):
    ws = ([ppl[k] if ppl[k].ndim == 2 else _row(ppl[k]) for k in A2_KEYS]
          + [paa[k] if paa[k].ndim == 2 else _row(paa[k]) for k in A2_KEYS])
    full = lambda a: pl.BlockSpec(a.shape, lambda i: (0,) * a.ndim)
    return pl.pallas_call(
        _a2s_body,
        grid=(B * T,),
        in_specs=[pl.BlockSpec((1, N, H), lambda i: (i, 0, 0)),
                  full(xpl),
                  pl.BlockSpec((1, N, KPL), lambda i: (i, 0, 0)),
                  pl.BlockSpec((1, N * KPL, H), lambda i: (i, 0, 0)),
                  pl.BlockSpec((1, N, KAA), lambda i: (i, 0, 0)),
                  pl.BlockSpec((1, N * KAA, H), lambda i: (i, 0, 0))]
                 + [full(a) for a in ws],
        out_specs=pl.BlockSpec((1, N, H), lambda i: (i, 0, 0)),
        out_shape=jax.ShapeDtypeStruct((B * T, N, H), jnp.float32),
    )(xs, xpl, knn_pl, rpl_rows, knn_aa, raa_rows, *ws)


SC_NC = 2
SC_NS = 16
SC_NW = SC_NC * SC_NS
ROWS_W = (B * N * T) // SC_NW


def _take16(v, idx):
    dn = lax.GatherDimensionNumbers(offset_dims=(), collapsed_slice_dims=(0,),
                                    start_index_map=(0,))
    return lax.gather(v, idx[:, None], dn, (1,),
                      mode=lax.GatherScatterMode.PROMISE_IN_BOUNDS)


def _lanes_min(v):
    lanes = jnp.arange(16, dtype=jnp.int32)
    for sh in (8, 4, 2, 1):
        v = jnp.minimum(v, _take16(v, lanes ^ sh))
    return v


def _sc_topk_rows(src_ref, dst_ref, nvec):
    lanes = jnp.arange(16, dtype=jnp.int32)
    big = jnp.float32(3.0e38)

    def row_body(r, c0):
        ks = [src_ref[r, pl.ds(16 * c, 16)] for c in range(nvec)]

        def sel_body(k, carry):
            kv = list(carry[:nvec])
            res = carry[nvec]
            m = kv[0]
            for c in range(1, nvec):
                m = jnp.minimum(m, kv[c])
            kmin = _lanes_min(m)
            cand = None
            for c in range(nvec):
                cc = jnp.where(kv[c] == kmin, lanes + 16 * c, jnp.int32(32767))
                cand = cc if cand is None else jnp.minimum(cand, cc)
            imin = _lanes_min(cand)
            res = jnp.where(lanes == k, imin, res)
            for c in range(nvec):
                kv[c] = jnp.where(lanes + 16 * c == imin, big, kv[c])
            return tuple(kv) + (res,)

        init = tuple(ks) + (jnp.zeros((16,), jnp.int32),)
        out = lax.fori_loop(0, KPL, sel_body, init)
        dst_ref[r, :] = out[nvec]
        return c0

    lax.fori_loop(0, ROWS_W, row_body, 0)


def _knn_sc(dpl, daa):
    mesh = plsc.VectorSubcoreMesh(core_axis_name="c", subcore_axis_name="s")

    @functools.partial(
        pl.kernel, mesh=mesh,
        out_type=(jax.ShapeDtypeStruct((SC_NW, ROWS_W, KPL), jnp.int32),
                  jax.ShapeDtypeStruct((SC_NW, ROWS_W, KAA), jnp.int32)),
        scratch_types=[pltpu.VMEM((ROWS_W, M), jnp.float32),
                       pltpu.VMEM((ROWS_W, N), jnp.float32),
                       pltpu.VMEM((ROWS_W, KPL), jnp.int32),
                       pltpu.VMEM((ROWS_W, KAA), jnp.int32)],
    )
    def knn(dpl_hbm, daa_hbm, opl_hbm, oaa_hbm, dpl_v, daa_v, opl_v, oaa_v):
        wid = lax.axis_index("s") * SC_NC + lax.axis_index("c")
        pltpu.sync_copy(dpl_hbm.at[wid], dpl_v)
        pltpu.sync_copy(daa_hbm.at[wid], daa_v)
        _sc_topk_rows(dpl_v, opl_v, M // 16)
        _sc_topk_rows(daa_v, oaa_v, N // 16)
        pltpu.sync_copy(opl_v, opl_hbm.at[wid])
        pltpu.sync_copy(oaa_v, oaa_hbm.at[wid])

    opl, oaa = knn(dpl.reshape(SC_NW, ROWS_W, M), daa.reshape(SC_NW, ROWS_W, N))
    return opl.reshape(B * N * T, KPL), oaa.reshape(B * T * N, KAA)


def _ang(ctr, nbr):
    return jnp.arctan2(ctr[..., 0] * nbr[..., 1] - ctr[..., 1] * nbr[..., 0],
                       jnp.sum(ctr * nbr, axis=-1))


def _wrap(a):
    return (a + jnp.pi) % (2.0 * jnp.pi) - jnp.pi


def kernel(agent_poses_his, agent_vels_his, pl_poses, x_pl, agent_classes, params):
    prm = params
    poses = agent_poses_his
    pos = poses[..., :2]
    hd = poses[..., 2]
    motion = jnp.concatenate(
        [jnp.zeros((B, N, 1, 2), poses.dtype), pos[:, :, 1:] - pos[:, :, :-1]], axis=2)
    hvec = jnp.stack([jnp.cos(hd), jnp.sin(hd)], axis=-1)
    cont = jnp.stack([
        jnp.linalg.norm(motion, axis=-1),
        _ang(hvec, motion),
        jnp.linalg.norm(agent_vels_his, axis=-1),
        _ang(hvec, agent_vels_his),
    ], axis=-1)
    cat = prm['type_emb'][agent_classes][:, :, None, :]
    cat_rows = jnp.broadcast_to(cat, (B, N, T, H)).reshape(B * N * T, H)
    x = _fourier(prm['x_a'], cont.reshape(B * N * T, 4), cat_rows).reshape(B, N, T, H)

    plp = pl_poses[..., :2]
    plh = pl_poses[..., 2]
    rel_all = pos[:, :, :, None, :] - plp[:, None, None, :, :]
    d2_pl = jnp.sum(rel_all * rel_all, axis=-1)
    pos_t = jnp.transpose(pos, (0, 2, 1, 3))
    rel_aa_all = pos_t[:, :, :, None, :] - pos_t[:, :, None, :, :]
    d2_aa = jnp.sum(rel_aa_all * rel_aa_all, axis=-1)
    idx_flat, idx2_flat = _knn_sc(jnp.sqrt(d2_pl).reshape(B * N * T, M),
                                  jnp.sqrt(d2_aa).reshape(B * T * N, N))
    idx = idx_flat.reshape(B, N, T, KPL)
    idx2 = idx2_flat.reshape(B, T, N, KAA)
    knn_pl = jnp.transpose(idx, (0, 2, 1, 3)).reshape(B * T, N, KPL)
    knn_tn = knn_pl.reshape(B, T, N, KPL)
    hd_t = jnp.transpose(hd, (0, 2, 1))
    hvec_t = jnp.transpose(hvec, (0, 2, 1, 3))
    plp_g = plp[0][knn_tn]
    plh_g = plh[0][knn_tn]
    rel_g = pos_t[:, :, :, None, :] - plp_g
    d_g = jnp.linalg.norm(rel_g, axis=-1)
    r_pl_raw = jnp.stack([
        d_g,
        _ang(hvec_t[:, :, :, None, :], rel_g),
        _wrap(plh_g - hd_t[..., None]),
    ], axis=-1)
    rpl_rows = _fourier(prm['r_a2pl'], r_pl_raw.reshape(-1, 3)).reshape(B * T, N * KPL, H)

    knn_aa = idx2.reshape(B * T, N, KAA)
    pos_g = jnp.take_along_axis(
        jnp.broadcast_to(pos_t[:, :, None, :, :], (B, T, N, N, 2)), idx2[..., None], axis=3)
    hd_g = jnp.take_along_axis(
        jnp.broadcast_to(hd_t[:, :, None, :], (B, T, N, N)), idx2, axis=3)
    rel_g2 = pos_t[:, :, :, None, :] - pos_g
    d_g2 = jnp.linalg.norm(rel_g2, axis=-1)
    r_aa_raw = jnp.stack([
        d_g2,
        _ang(hvec_t[:, :, :, None, :], rel_g2),
        _wrap(hd_t[..., None] - hd_g),
    ], axis=-1)
    raa_rows = _fourier(prm['r_a2a'], r_aa_raw.reshape(-1, 3)).reshape(B * T, N * KAA, H)

    rel_t = pos[:, :, :, None, :] - pos[:, :, None, :, :]
    rel_hv = hvec[:, :, :, None, :] - hvec[:, :, None, :, :]
    r_t_raw3 = jnp.stack([
        jnp.linalg.norm(rel_t, axis=-1),
        _ang(rel_hv, rel_t),
        _wrap(hd[:, :, :, None] - hd[:, :, None, :]),
    ], axis=-1)
    raw3 = jnp.pad(r_t_raw3, ((0, 0), (0, 0), (0, 0), (0, TP - T), (0, 0))
                   ).reshape(B * N, T * TP, 3)
    tib_x = (jnp.arange(T)[:, None] - jnp.arange(TP)[None, :]
             ).astype(jnp.float32).reshape(T * TP, 1)
    tib = _fourier_one_branch(prm['r_t'], tib_x, 3)
    rt2 = _fourier_grouped(prm['r_t'], raw3, tib, 3)

    xpl_t = x_pl[0]
    for i in range(NL):
        xt = jnp.pad(x.reshape(B * N, T, H), ((0, 0), (0, TP - T), (0, 0)))
        xt = _a2t(prm['a2t'][i], xt, rt2)[:, :T]
        xs = jnp.transpose(xt.reshape(B, N, T, H), (0, 2, 1, 3)).reshape(B * T, N, H)
        xs = _a2s(prm['a2pl'][i], prm['a2a'][i], xs, xpl_t,
                  knn_pl, rpl_rows, knn_aa, raa_rows)
        x = jnp.transpose(xs.reshape(B, T, N, H), (0, 2, 1, 3))
    return x
